# baseline (speedup 1.0000x reference)
import jax
import jax.numpy as jnp
from jax import lax
from jax.experimental import pallas as pl
from jax.experimental.pallas import tpu as pltpu
from jax.experimental.pallas import tpu_sc as plsc

N = 100000
D = 128
NTC = 60000
BTC = 2000
SC_START = NTC
W = 80
NCHUNK = (N - SC_START) // W
NW = 32
BASE_CH = NCHUNK // NW
EXTRA = NCHUNK % NW
K = 5
SUP = ((BASE_CH + 1 + K - 1) // K + 1) // 2 * 2
IDX_MAX = (BASE_CH + 1) * W


def _sc_fill(idx, table):
    mesh = plsc.VectorSubcoreMesh(core_axis_name="c", subcore_axis_name="s")

    @pl.kernel(
        out_type=jax.ShapeDtypeStruct((N, D), table.dtype),
        mesh=mesh,
        scratch_types=[
            pltpu.VMEM_SHARED((28, D), jnp.float32),
            pltpu.VMEM((IDX_MAX,), jnp.int32),
            pltpu.VMEM((K * W, D), jnp.float32),
            pltpu.VMEM((K * W, D), jnp.float32),
            pltpu.SemaphoreType.DMA,
            pltpu.SemaphoreType.DMA,
            pltpu.SemaphoreType.DMA,
            pltpu.SemaphoreType.DMA,
        ],
    )
    def gather_kernel(idx_hbm, table_hbm, out_hbm, table_sh, idx_v, buf0, buf1,
                      g0, g1, s0, s1):
        sid = lax.axis_index("s")
        wid = sid * 2 + lax.axis_index("c")

        @pl.when(sid == 0)
        def _():
            pltpu.sync_copy(table_hbm, table_sh)

        plsc.subcore_barrier()
        start_chunk = BASE_CH * wid + jnp.minimum(wid, EXTRA)
        n = jnp.where(wid < EXTRA, BASE_CH + 1, BASE_CH)
        row0 = SC_START + start_chunk * W

        @pl.when(wid < EXTRA)
        def _():
            pltpu.sync_copy(idx_hbm.at[pl.ds(row0, (BASE_CH + 1) * W)],
                            idx_v.at[pl.ds(0, (BASE_CH + 1) * W)])

        @pl.when(wid >= EXTRA)
        def _():
            pltpu.sync_copy(idx_hbm.at[pl.ds(row0, BASE_CH * W)],
                            idx_v.at[pl.ds(0, BASE_CH * W)])

        def gather_desc(t, buf, b, sem):
            return pltpu.make_async_copy(
                table_sh.at[idx_v.at[pl.ds(t * W, W)]],
                buf.at[pl.ds(b * W, W)], sem)

        def store_desc(t, buf, b, sem):
            return pltpu.make_async_copy(
                buf.at[pl.ds(b * W, W)],
                out_hbm.at[pl.ds(row0 + t * W, W)], sem)

        def fire_gathers(s, buf, sem):
            for b in range(K):
                t = s * K + b

                @pl.when(t < n)
                def _():
                    gather_desc(t, buf, b, sem).start()

        def wait_gathers(s, buf, sem):
            for b in range(K):
                t = s * K + b

                @pl.when(t < n)
                def _():
                    gather_desc(t, buf, b, sem).wait()

        def super_desc(s, buf, sem):
            return pltpu.make_async_copy(
                buf, out_hbm.at[pl.ds(row0 + s * K * W, K * W)], sem)

        def fire_stores(s, buf, sem):
            @pl.when(s * K + K <= n)
            def _():
                super_desc(s, buf, sem).start()

            @pl.when(s * K + K > n)
            def _():
                for b in range(K):
                    t = s * K + b

                    @pl.when(t < n)
                    def _():
                        store_desc(t, buf, b, sem).start()

        def wait_stores(s, buf, sem):
            @pl.when(s * K + K <= n)
            def _():
                super_desc(s, buf, sem).wait()

            @pl.when(s * K + K > n)
            def _():
                for b in range(K):
                    t = s * K + b

                    @pl.when(t < n)
                    def _():
                        store_desc(t, buf, b, sem).wait()

        fire_gathers(0, buf0, g0)
        fire_gathers(1, buf1, g1)

        @pl.loop(0, SUP, step=2)
        def _(s):
            wait_gathers(s, buf0, g0)
            fire_stores(s, buf0, s0)
            wait_gathers(s + 1, buf1, g1)
            wait_stores(s, buf0, s0)
            fire_gathers(s + 2, buf0, g0)
            fire_stores(s + 1, buf1, s1)
            wait_stores(s + 1, buf1, s1)
            fire_gathers(s + 3, buf1, g1)

    return gather_kernel(idx, table)


def _tc_body(idx_ref, table_ref, _, out_ref):
    idx = idx_ref[...]
    types = lax.broadcasted_iota(jnp.int32, (BTC, 32), 1)
    onehot = (idx == types).astype(jnp.float32)
    out_ref[...] = lax.dot_general(
        onehot, table_ref[...], (((1,), (0,)), ((), ())),
        precision=lax.Precision.HIGHEST,
        preferred_element_type=jnp.float32)


def kernel(x, table):
    idx = x.reshape(N)
    out_sc = _sc_fill(idx, table)

    table_pad = jnp.zeros((32, D), jnp.float32).at[:28].set(table)
    out = pl.pallas_call(
        _tc_body,
        grid=(NTC // BTC,),
        in_specs=[
            pl.BlockSpec((BTC, 1), lambda i: (i, 0)),
            pl.BlockSpec((32, D), lambda i: (0, 0)),
            pl.BlockSpec(memory_space=pl.ANY),
        ],
        out_specs=pl.BlockSpec((BTC, D), lambda i: (i, 0)),
        out_shape=jax.ShapeDtypeStruct((N, D), jnp.float32),
        input_output_aliases={2: 0},
    )(x[:NTC], table_pad, out_sc)
    return out


# speedup vs baseline: 1.9321x; 1.9321x over previous
import jax
import jax.numpy as jnp
from jax import lax
from jax.experimental import pallas as pl
from jax.experimental.pallas import tpu as pltpu
from jax.experimental.pallas import tpu_sc as plsc

N = 100000
D = 128
W = 80
NCHUNK = N // W
NW = 32
BASE_CH = NCHUNK // NW
EXTRA = NCHUNK % NW
K = 2
SUP = (BASE_CH + 1 + K - 1) // K
IDX_MAX = (BASE_CH + 1) * W


def kernel(x, table):
    idx = x.reshape(N).astype(jnp.int32)
    mesh = plsc.VectorSubcoreMesh(core_axis_name="c", subcore_axis_name="s")

    @pl.kernel(
        out_type=jax.ShapeDtypeStruct((N, D), table.dtype),
        mesh=mesh,
        scratch_types=[
            pltpu.VMEM_SHARED((28, D), jnp.float32),
            pltpu.VMEM((IDX_MAX,), jnp.int32),
            pltpu.VMEM((K * W, D), jnp.float32),
            pltpu.VMEM((K * W, D), jnp.float32),
            pltpu.SemaphoreType.DMA,
            pltpu.SemaphoreType.DMA,
            pltpu.SemaphoreType.DMA,
            pltpu.SemaphoreType.DMA,
        ],
    )
    def gather_kernel(idx_hbm, table_hbm, out_hbm, table_sh, idx_v, buf0, buf1,
                      g0, g1, s0, s1):
        sid = lax.axis_index("s")
        wid = sid * 2 + lax.axis_index("c")

        start_chunk = BASE_CH * wid + jnp.minimum(wid, EXTRA)
        n = jnp.where(wid < EXTRA, BASE_CH + 1, BASE_CH)
        row0 = start_chunk * W

        @pl.when(sid == 0)
        def _():
            pltpu.make_async_copy(table_hbm, table_sh, g0).start()

        @pl.when(wid < EXTRA)
        def _():
            pltpu.sync_copy(idx_hbm.at[pl.ds(row0, (BASE_CH + 1) * W)],
                            idx_v.at[pl.ds(0, (BASE_CH + 1) * W)])

        @pl.when(wid >= EXTRA)
        def _():
            pltpu.sync_copy(idx_hbm.at[pl.ds(row0, BASE_CH * W)],
                            idx_v.at[pl.ds(0, BASE_CH * W)])

        @pl.when(sid == 0)
        def _():
            pltpu.make_async_copy(table_hbm, table_sh, g0).wait()

        plsc.subcore_barrier()

        def gather_desc(t, buf, b, sem):
            return pltpu.make_async_copy(
                table_sh.at[idx_v.at[pl.ds(t * W, W)]],
                buf.at[pl.ds(b * W, W)], sem)

        def store_desc(t, buf, b, sem):
            return pltpu.make_async_copy(
                buf.at[pl.ds(b * W, W)],
                out_hbm.at[pl.ds(row0 + t * W, W)], sem)

        def fire_gathers(s, buf, sem):
            for b in range(K):
                t = s * K + b

                @pl.when(t < n)
                def _():
                    gather_desc(t, buf, b, sem).start()

        def wait_gathers(s, buf, sem):
            for b in range(K):
                t = s * K + b

                @pl.when(t < n)
                def _():
                    gather_desc(t, buf, b, sem).wait()

        def super_desc(s, buf, sem):
            return pltpu.make_async_copy(
                buf, out_hbm.at[pl.ds(row0 + s * K * W, K * W)], sem)

        def fire_stores(s, buf, sem):
            @pl.when(s * K + K <= n)
            def _():
                super_desc(s, buf, sem).start()

            @pl.when(s * K + K > n)
            def _():
                for b in range(K):
                    t = s * K + b

                    @pl.when(t < n)
                    def _():
                        store_desc(t, buf, b, sem).start()

        def wait_stores(s, buf, sem):
            @pl.when(s * K + K <= n)
            def _():
                super_desc(s, buf, sem).wait()

            @pl.when(s * K + K > n)
            def _():
                for b in range(K):
                    t = s * K + b

                    @pl.when(t < n)
                    def _():
                        store_desc(t, buf, b, sem).wait()

        fire_gathers(0, buf0, g0)
        fire_gathers(1, buf1, g1)

        @pl.loop(0, SUP, step=2)
        def _(s):
            wait_gathers(s, buf0, g0)
            fire_stores(s, buf0, s0)
            wait_gathers(s + 1, buf1, g1)
            wait_stores(s, buf0, s0)
            fire_gathers(s + 2, buf0, g0)
            fire_stores(s + 1, buf1, s1)
            wait_stores(s + 1, buf1, s1)
            fire_gathers(s + 3, buf1, g1)

    return gather_kernel(idx, table)
